# Initial kernel scaffold; baseline (speedup 1.0000x reference)
import jax
import jax.numpy as jnp
from jax.experimental import pallas as pl


def kernel(x_num, x_cat, tables, W1, b1, g1, be1, W2, b2, g2, be2, W3, b3, g3, be3, W4, b4):
    raise NotImplementedError("write your pallas kernel here")


# speedup vs baseline: 7.7843x; 7.7843x over previous
import functools

import jax
import jax.numpy as jnp
from jax import lax
from jax.experimental import pallas as pl
from jax.experimental.pallas import tpu as pltpu
from jax.experimental.pallas import tpu_sc as plsc

NUM_FIELDS = 26
VOCAB = 100000
EMB = 32
B = 16384
NUM_NUM = 13
EPS = 1e-5

NC = 2
NS = 16
NW = NC * NS
TOTAL_ROWS = B * NUM_FIELDS
ROWS_PER_W = TOTAL_ROWS // NW
CHUNK = 1664
NCHUNK = ROWS_PER_W // CHUNK

@functools.cache
def _make_sc_gather():
    mesh = plsc.VectorSubcoreMesh(
        core_axis_name="c", subcore_axis_name="s", num_cores=NC, num_subcores=NS
    )

    @functools.partial(
        pl.kernel,
        out_type=jax.ShapeDtypeStruct((TOTAL_ROWS, EMB), jnp.float32),
        mesh=mesh,
        scratch_types=[
            pltpu.VMEM((CHUNK,), jnp.int32),
            pltpu.VMEM((CHUNK, EMB), jnp.float32),
            pltpu.SemaphoreType.DMA,
        ],
        compiler_params=pltpu.CompilerParams(use_tc_tiling_on_sc=False),
    )
    def _sc_gather(tab_hbm, idx_hbm, out_hbm, idx_v, rows_v, sem):
        wid = lax.axis_index("s") * NC + lax.axis_index("c")
        base = wid * ROWS_PER_W
        for j in range(NCHUNK):
            off = base + j * CHUNK
            pltpu.sync_copy(idx_hbm.at[pl.ds(off, CHUNK)], idx_v)
            pltpu.async_copy(tab_hbm.at[idx_v], rows_v, sem).wait()
            pltpu.sync_copy(rows_v, out_hbm.at[pl.ds(off, CHUNK)])

    return _sc_gather


_BM = 512


def _mlp_body(xn, xe, w1n, w1e, b1, g1, be1, w2, b2, g2, be2,
              w3, b3, g3, be3, w4, b4, out):
    inv = 1.0 / (1.0 + EPS) ** 0.5
    h = jnp.dot(xn[...], w1n[...], preferred_element_type=jnp.float32)
    h = h + jnp.dot(xe[...], w1e[...], preferred_element_type=jnp.float32)
    h = jnp.maximum(h + b1[...], 0.0) * (g1[...] * inv) + be1[...]
    h = jnp.dot(h, w2[...], preferred_element_type=jnp.float32)
    h = jnp.maximum(h + b2[...], 0.0) * (g2[...] * inv) + be2[...]
    h = jnp.dot(h, w3[...], preferred_element_type=jnp.float32)
    h = jnp.maximum(h + b3[...], 0.0) * (g3[...] * inv) + be3[...]
    out[...] = jnp.dot(h, w4[...], preferred_element_type=jnp.float32) + b4[...]


def _mlp(xn, xe, w1n, w1e, b1, g1, be1, w2, b2, g2, be2, w3, b3, g3, be3, w4, b4):
    full = lambda r, c: pl.BlockSpec((r, c), lambda i: (0, 0))
    return pl.pallas_call(
        _mlp_body,
        grid=(B // _BM,),
        in_specs=[
            pl.BlockSpec((_BM, NUM_NUM), lambda i: (i, 0)),
            pl.BlockSpec((_BM, NUM_FIELDS * EMB), lambda i: (i, 0)),
            full(NUM_NUM, 1024), full(NUM_FIELDS * EMB, 1024),
            full(1, 1024), full(1, 1024), full(1, 1024),
            full(1024, 512), full(1, 512), full(1, 512), full(1, 512),
            full(512, 256), full(1, 256), full(1, 256), full(1, 256),
            full(256, 1), full(1, 1),
        ],
        out_specs=pl.BlockSpec((_BM, 1), lambda i: (i, 0)),
        out_shape=jax.ShapeDtypeStruct((B, 1), jnp.float32),
        compiler_params=pltpu.CompilerParams(
            dimension_semantics=("arbitrary",)
        ),
    )(xn, xe, w1n, w1e, b1, g1, be1, w2, b2, g2, be2, w3, b3, g3, be3, w4, b4)


def kernel(x_num, x_cat, tables, W1, b1, g1, be1, W2, b2, g2, be2,
           W3, b3, g3, be3, W4, b4):
    tab_flat = tables.reshape(NUM_FIELDS * VOCAB, EMB)
    offs = (jnp.arange(NUM_FIELDS, dtype=jnp.int32) * VOCAB)[None, :]
    idx_flat = (x_cat + offs).reshape(TOTAL_ROWS)
    emb = _make_sc_gather()(tab_flat, idx_flat).reshape(B, NUM_FIELDS * EMB)
    r2 = lambda v: v.reshape(1, -1)
    out = _mlp(x_num, emb, W1[:NUM_NUM], W1[NUM_NUM:], r2(b1), r2(g1), r2(be1),
               W2, r2(b2), r2(g2), r2(be2), W3, r2(b3), r2(g3), r2(be3),
               W4, r2(b4))
    return out[:, 0]


# speedup vs baseline: 20.7271x; 2.6627x over previous
import functools

import jax
import jax.numpy as jnp
from jax import lax
from jax.experimental import pallas as pl
from jax.experimental.pallas import tpu as pltpu
from jax.experimental.pallas import tpu_sc as plsc

NUM_FIELDS = 26
VOCAB = 100000
EMB = 32
B = 16384
NUM_NUM = 13
EPS = 1e-5

NC = 2
NS = 16
NW = NC * NS
ROWS = NUM_FIELDS * EMB
RPW = ROWS // NW
HALF = B // 2
LANES = 16


@functools.cache
def _make_sc_lookup():
    mesh = plsc.VectorSubcoreMesh(
        core_axis_name="c", subcore_axis_name="s", num_cores=NC, num_subcores=NS
    )

    @functools.partial(
        pl.kernel,
        out_type=jax.ShapeDtypeStruct((ROWS, B), jnp.float32),
        mesh=mesh,
        scratch_types=[
            pltpu.VMEM((VOCAB,), jnp.float32),
            pltpu.VMEM((B,), jnp.int32),
            pltpu.VMEM((HALF,), jnp.float32),
        ],
        compiler_params=pltpu.CompilerParams(
            use_tc_tiling_on_sc=True, needs_layout_passes=False
        ),
    )
    def _sc_lookup(tabT_hbm, xcatT_hbm, out_hbm, row_v, idx_v, out_v):
        wid = lax.axis_index("s") * NC + lax.axis_index("c")
        r0 = wid * RPW
        for j in range(RPW):
            r = r0 + j
            f = r // EMB
            pltpu.sync_copy(xcatT_hbm.at[f], idx_v)
            pltpu.sync_copy(tabT_hbm.at[r], row_v)
            for h in range(2):
                def body(i, _, h=h):
                    idxv = idx_v[pl.ds(h * HALF + i * LANES, LANES)]
                    out_v[pl.ds(i * LANES, LANES)] = plsc.load_gather(
                        row_v, [idxv]
                    )
                    return 0
                lax.fori_loop(0, HALF // LANES, body, 0, unroll=8)
                pltpu.sync_copy(out_v, out_hbm.at[r, pl.ds(h * HALF, HALF)])

    return _sc_lookup


_BN = 1024


def _mlp_body(xnT, eT, w1nT, w1eT, b1, g1, be1, w2T, b2, g2, be2,
              w3T, b3, g3, be3, w4T, b4, out):
    inv = 1.0 / (1.0 + EPS) ** 0.5
    h = jnp.dot(w1eT[...], eT[...], preferred_element_type=jnp.float32)
    h = h + jnp.dot(w1nT[...], xnT[...], preferred_element_type=jnp.float32)
    h = jnp.maximum(h + b1[...], 0.0) * (g1[...] * inv) + be1[...]
    h = jnp.dot(w2T[...], h, preferred_element_type=jnp.float32)
    h = jnp.maximum(h + b2[...], 0.0) * (g2[...] * inv) + be2[...]
    h = jnp.dot(w3T[...], h, preferred_element_type=jnp.float32)
    h = jnp.maximum(h + b3[...], 0.0) * (g3[...] * inv) + be3[...]
    out[...] = jnp.dot(w4T[...], h, preferred_element_type=jnp.float32) + b4[...]


def _mlp(xnT, eT, w1nT, w1eT, b1, g1, be1, w2T, b2, g2, be2,
         w3T, b3, g3, be3, w4T, b4):
    full = lambda r, c: pl.BlockSpec((r, c), lambda i: (0, 0))
    col = lambda r: pl.BlockSpec((r, _BN), lambda i: (0, i))
    return pl.pallas_call(
        _mlp_body,
        grid=(B // _BN,),
        in_specs=[
            col(NUM_NUM), col(ROWS),
            full(1024, NUM_NUM), full(1024, ROWS),
            full(1024, 1), full(1024, 1), full(1024, 1),
            full(512, 1024), full(512, 1), full(512, 1), full(512, 1),
            full(256, 512), full(256, 1), full(256, 1), full(256, 1),
            full(1, 256), full(1, 1),
        ],
        out_specs=col(1),
        out_shape=jax.ShapeDtypeStruct((1, B), jnp.float32),
        compiler_params=pltpu.CompilerParams(
            dimension_semantics=("arbitrary",)
        ),
    )(xnT, eT, w1nT, w1eT, b1, g1, be1, w2T, b2, g2, be2,
      w3T, b3, g3, be3, w4T, b4)


def kernel(x_num, x_cat, tables, W1, b1, g1, be1, W2, b2, g2, be2,
           W3, b3, g3, be3, W4, b4):
    tabT = tables.transpose(0, 2, 1).reshape(ROWS, VOCAB)
    xcatT = x_cat.T
    embT = _make_sc_lookup()(tabT, xcatT)
    c = lambda v: v.reshape(-1, 1)
    out = _mlp(x_num.T, embT,
               W1[:NUM_NUM].T, W1[NUM_NUM:].T, c(b1), c(g1), c(be1),
               W2.T, c(b2), c(g2), c(be2), W3.T, c(b3), c(g3), c(be3),
               W4.T, c(b4))
    return out[0]


# speedup vs baseline: 22.7935x; 1.0997x over previous
import functools

import jax
import jax.numpy as jnp
from jax import lax
from jax.experimental import pallas as pl
from jax.experimental.pallas import tpu as pltpu
from jax.experimental.pallas import tpu_sc as plsc

NUM_FIELDS = 26
VOCAB = 100000
EMB = 32
B = 16384
NUM_NUM = 13
EPS = 1e-5

NC = 2
NS = 16
NW = NC * NS
ROWS = NUM_FIELDS * EMB
RPW = ROWS // NW
NQ = 4
QTR = B // NQ
LANES = 16


@functools.cache
def _make_sc_lookup():
    mesh = plsc.VectorSubcoreMesh(
        core_axis_name="c", subcore_axis_name="s", num_cores=NC, num_subcores=NS
    )

    @functools.partial(
        pl.kernel,
        out_type=jax.ShapeDtypeStruct((ROWS, B), jnp.float32),
        mesh=mesh,
        scratch_types=[
            pltpu.VMEM((VOCAB,), jnp.float32),
            pltpu.VMEM((B,), jnp.int32),
            pltpu.VMEM((QTR,), jnp.float32),
            pltpu.VMEM((QTR,), jnp.float32),
            pltpu.SemaphoreType.DMA,
            pltpu.SemaphoreType.DMA,
        ],
        compiler_params=pltpu.CompilerParams(
            use_tc_tiling_on_sc=True, needs_layout_passes=False
        ),
    )
    def _sc_lookup(tabT_hbm, xcatT_hbm, out_hbm, row_v, idx_v,
                   out_v0, out_v1, sem0, sem1):
        wid = lax.axis_index("s") * NC + lax.axis_index("c")
        r0 = wid * RPW
        outs = (out_v0, out_v1)
        sems = (sem0, sem1)
        desc = [None, None]
        for j in range(RPW):
            r = r0 + j
            f = r // EMB
            if j == 0:
                pltpu.sync_copy(xcatT_hbm.at[f], idx_v)
            else:
                fprev = (r - 1) // EMB

                @pl.when(f != fprev)
                def _():
                    pltpu.sync_copy(xcatT_hbm.at[f], idx_v)

            pltpu.sync_copy(tabT_hbm.at[r], row_v)
            for q in range(NQ):
                k = (j * NQ + q) % 2
                if desc[k] is not None:
                    desc[k].wait()
                ov = outs[k]

                def body(i, _, q=q, ov=ov):
                    idxv = idx_v[pl.ds(q * QTR + i * LANES, LANES)]
                    ov[pl.ds(i * LANES, LANES)] = plsc.load_gather(
                        row_v, [idxv]
                    )
                    return 0

                lax.fori_loop(0, QTR // LANES, body, 0, unroll=8)

                desc[k] = pltpu.async_copy(
                    ov, out_hbm.at[r, pl.ds(q * QTR, QTR)], sems[k]
                )
        for d in desc:
            if d is not None:
                d.wait()

    return _sc_lookup


_BN = 1024


def _mlp_body(xnT, eT, w1nT, w1eT, b1, g1, be1, w2T, b2, g2, be2,
              w3T, b3, g3, be3, w4T, b4, out):
    inv = 1.0 / (1.0 + EPS) ** 0.5
    h = jnp.dot(w1eT[...], eT[...], preferred_element_type=jnp.float32)
    h = h + jnp.dot(w1nT[...], xnT[...], preferred_element_type=jnp.float32)
    h = jnp.maximum(h + b1[...], 0.0) * (g1[...] * inv) + be1[...]
    h = jnp.dot(w2T[...], h, preferred_element_type=jnp.float32)
    h = jnp.maximum(h + b2[...], 0.0) * (g2[...] * inv) + be2[...]
    h = jnp.dot(w3T[...], h, preferred_element_type=jnp.float32)
    h = jnp.maximum(h + b3[...], 0.0) * (g3[...] * inv) + be3[...]
    out[...] = jnp.dot(w4T[...], h, preferred_element_type=jnp.float32) + b4[...]


def _mlp(xnT, eT, w1nT, w1eT, b1, g1, be1, w2T, b2, g2, be2,
         w3T, b3, g3, be3, w4T, b4):
    full = lambda r, c: pl.BlockSpec((r, c), lambda i: (0, 0))
    col = lambda r: pl.BlockSpec((r, _BN), lambda i: (0, i))
    return pl.pallas_call(
        _mlp_body,
        grid=(B // _BN,),
        in_specs=[
            col(NUM_NUM), col(ROWS),
            full(1024, NUM_NUM), full(1024, ROWS),
            full(1024, 1), full(1024, 1), full(1024, 1),
            full(512, 1024), full(512, 1), full(512, 1), full(512, 1),
            full(256, 512), full(256, 1), full(256, 1), full(256, 1),
            full(1, 256), full(1, 1),
        ],
        out_specs=col(1),
        out_shape=jax.ShapeDtypeStruct((1, B), jnp.float32),
        compiler_params=pltpu.CompilerParams(
            dimension_semantics=("arbitrary",)
        ),
    )(xnT, eT, w1nT, w1eT, b1, g1, be1, w2T, b2, g2, be2,
      w3T, b3, g3, be3, w4T, b4)


def kernel(x_num, x_cat, tables, W1, b1, g1, be1, W2, b2, g2, be2,
           W3, b3, g3, be3, W4, b4):
    tabT = tables.transpose(0, 2, 1).reshape(ROWS, VOCAB)
    xcatT = x_cat.T
    embT = _make_sc_lookup()(tabT, xcatT)
    c = lambda v: v.reshape(-1, 1)
    out = _mlp(x_num.T, embT,
               W1[:NUM_NUM].T, W1[NUM_NUM:].T, c(b1), c(g1), c(be1),
               W2.T, c(b2), c(g2), c(be2), W3.T, c(b3), c(g3), c(be3),
               W4.T, c(b4))
    return out[0]
